# Initial kernel scaffold; baseline (speedup 1.0000x reference)
import jax
import jax.numpy as jnp
from jax.experimental import pallas as pl


def kernel(score, logits, regress, anchors):
    raise NotImplementedError("write your pallas kernel here")


# speedup vs baseline: 22.0322x; 22.0322x over previous
import math

import jax
import jax.numpy as jnp
import numpy as np
from jax.experimental import pallas as pl
from jax.experimental.pallas import tpu as pltpu

_N_BOX = 20000
_NPAD = 20480
_C = 8
_T = 100
_NCAND = _C * _T
_IOU_THR = 0.3
_SCORE_THR = 0.7
_SIGMA = 0.5
_CLIP_RATIO = 16.0 / 1000.0


def _nms_body(score_ref, logits_ref, regress_ref, anchors_ref,
              logit_out, boxes_out, cscore_out, cidx_out,
              s_ref, cat_ref, area_ref):
    f32 = jnp.float32
    logitT = score_ref[...] * logits_ref[...]
    logit_out[...] = logitT

    ax1 = anchors_ref[0:1, :]
    ay1 = anchors_ref[1:2, :]
    ax2 = anchors_ref[2:3, :]
    ay2 = anchors_ref[3:4, :]
    ws = ax2 - ax1
    hs = ay2 - ay1
    cx = ax1 + 0.5 * ws
    cy = ay1 + 0.5 * hs
    dx = regress_ref[0:1, :]
    dy = regress_ref[1:2, :]
    mr = f32(abs(math.log(_CLIP_RATIO)))
    dw = jnp.clip(regress_ref[2:3, :], -mr, mr)
    dh = jnp.clip(regress_ref[3:4, :], -mr, mr)
    pcx = cx + dx * ws
    pcy = cy + dy * hs
    pw = ws * jnp.exp(dw)
    ph = hs * jnp.exp(dh)
    x1 = jnp.clip(pcx - 0.5 * pw, 0.0, 1.0)
    y1 = jnp.clip(pcy - 0.5 * ph, 0.0, 1.0)
    x2 = jnp.clip(pcx + 0.5 * pw, 0.0, 1.0)
    y2 = jnp.clip(pcy + 0.5 * ph, 0.0, 1.0)
    boxesT = jnp.concatenate([x1, y1, x2, y2], axis=0)
    boxes_out[...] = boxesT
    cat_ref[...] = jnp.concatenate([boxesT, logitT], axis=0)
    area_ref[...] = jnp.maximum(x2 - x1, 0.0) * jnp.maximum(y2 - y1, 0.0)

    m8 = jnp.max(logitT, axis=0, keepdims=True)
    validm = (m8 >= _SCORE_THR) & (logitT[0:1, :] < m8)
    row = jax.lax.broadcasted_iota(jnp.int32, (_C, _NPAD), 0)
    s0 = jnp.where(validm, logitT, 0.0)
    s0 = jnp.where(row == 0, f32(-1e9), s0)
    s_ref[...] = s0

    lane_f = jax.lax.broadcasted_iota(jnp.int32, (_C, _NPAD), 1).astype(f32)
    row8 = jax.lax.broadcasted_iota(jnp.int32, (_C, 1), 0)
    eye8 = (jax.lax.broadcasted_iota(jnp.int32, (_C, _C), 0)
            == jax.lax.broadcasted_iota(jnp.int32, (_C, _C), 1)).astype(f32)
    col128 = jax.lax.broadcasted_iota(jnp.int32, (1, 128), 1)
    cscore_out[...] = jnp.zeros((_C, 128), f32)
    cidx_out[...] = jnp.zeros((_C, 128), f32)

    def step(t, carry):
        s = s_ref[...]
        m = jnp.max(s, axis=1, keepdims=True)
        i_f = jnp.min(jnp.where(s == m, lane_f, f32(_NPAD)),
                      axis=1, keepdims=True)
        oh_b = lane_f == i_f
        oh_f = oh_b.astype(f32)
        g = jax.lax.dot_general(oh_f, cat_ref[...],
                                (((1,), (1,)), ((), ())),
                                preferred_element_type=f32,
                                precision=jax.lax.Precision.HIGHEST)
        bx1 = g[:, 0:1]
        by1 = g[:, 1:2]
        bx2 = g[:, 2:3]
        by2 = g[:, 3:4]
        cls = jnp.sum(g[:, 4:12] * eye8, axis=1, keepdims=True)
        cls = jnp.where(row8 == 0, f32(-1e9), cls)
        ix1 = jnp.maximum(bx1, cat_ref[0:1, :])
        iy1 = jnp.maximum(by1, cat_ref[1:2, :])
        ix2 = jnp.minimum(bx2, cat_ref[2:3, :])
        iy2 = jnp.minimum(by2, cat_ref[3:4, :])
        inter = jnp.maximum(ix2 - ix1, 0.0) * jnp.maximum(iy2 - iy1, 0.0)
        a0 = jnp.maximum(bx2 - bx1, 0.0) * jnp.maximum(by2 - by1, 0.0)
        union = a0 + area_ref[...] - inter
        iou = inter / jnp.maximum(union, f32(1e-9))
        w = jnp.where(iou <= _IOU_THR,
                      jnp.exp(-0.5 * iou * iou / f32(_SIGMA)), 0.0)
        s_ref[...] = jnp.where(oh_b, f32(-1.0), s * w)
        oh_t = (col128 == t).astype(f32)
        cscore_out[...] = cscore_out[...] + cls * oh_t
        cidx_out[...] = cidx_out[...] + i_f * oh_t
        return carry

    jax.lax.fori_loop(0, _T, step, 0)


def _topk_body(vcol_ref, vrow_ref, idx_ref, logit_ref, boxes_ref,
               ol_ref, op_ref):
    f32 = jnp.float32
    vcol = vcol_ref[...]
    vrow = vrow_ref[...]
    ri = jax.lax.broadcasted_iota(jnp.int32, (_NCAND, _NCAND), 0)
    ci = jax.lax.broadcasted_iota(jnp.int32, (_NCAND, _NCAND), 1)
    beats = (vcol > vrow) | ((vcol == vrow) & (ri < ci))
    rank = jnp.sum(beats.astype(f32), axis=0, keepdims=True)
    rr = jax.lax.broadcasted_iota(jnp.int32, (_T, _NCAND), 0).astype(f32)
    pr = (rr == rank).astype(f32)
    sel = jax.lax.dot_general(pr, idx_ref[...],
                              (((1,), (0,)), ((), ())),
                              preferred_element_type=f32,
                                precision=jax.lax.Precision.HIGHEST)
    sel_i = sel.astype(jnp.int32)
    li = jax.lax.broadcasted_iota(jnp.int32, (_T, _NPAD), 1)
    oh = (li == sel_i).astype(f32)
    ol_ref[...] = jax.lax.dot_general(oh, logit_ref[...],
                                      (((1,), (1,)), ((), ())),
                                      preferred_element_type=f32,
                                precision=jax.lax.Precision.HIGHEST)
    op_ref[...] = jax.lax.dot_general(oh, boxes_ref[...],
                                      (((1,), (1,)), ((), ())),
                                      preferred_element_type=f32,
                                precision=jax.lax.Precision.HIGHEST)


def _stage1(scoreT, logitsT, regressT, anchorsT):
    f32 = jnp.float32
    return pl.pallas_call(
        _nms_body,
        out_shape=[
            jax.ShapeDtypeStruct((_C, _NPAD), f32),
            jax.ShapeDtypeStruct((4, _NPAD), f32),
            jax.ShapeDtypeStruct((_C, 128), f32),
            jax.ShapeDtypeStruct((_C, 128), f32),
        ],
        scratch_shapes=[
            pltpu.VMEM((_C, _NPAD), f32),
            pltpu.VMEM((12, _NPAD), f32),
            pltpu.VMEM((1, _NPAD), f32),
        ],
    )(scoreT, logitsT, regressT, anchorsT)


def _stage2(v, ix, logitT, boxesT):
    f32 = jnp.float32
    return pl.pallas_call(
        _topk_body,
        out_shape=[
            jax.ShapeDtypeStruct((_T, _C), f32),
            jax.ShapeDtypeStruct((_T, 4), f32),
        ],
    )(v[:, None], v[None, :], ix[:, None], logitT, boxesT)


def kernel(score, logits, regress, anchors):
    f32 = jnp.float32
    pad = _NPAD - _N_BOX
    scoreT = jnp.pad(score[0, :, 0][None, :].astype(f32), ((0, 0), (0, pad)))
    logitsT = jnp.pad(logits[0].T.astype(f32), ((0, 0), (0, pad)))
    regressT = jnp.pad(regress[0].T.astype(f32), ((0, 0), (0, pad)))
    anchorsT = jnp.pad(anchors.T.astype(f32), ((0, 0), (0, pad)))
    logitT, boxesT, cscore, cidx = _stage1(scoreT, logitsT, regressT, anchorsT)
    v = cscore[:, :_T].reshape(-1)
    ix = cidx[:, :_T].reshape(-1)
    out_logit, out_prop = _stage2(v, ix, logitT, boxesT)
    return out_logit[None], out_prop[None]
